# baseline (speedup 1.0000x reference)
import functools

import jax
import jax.numpy as jnp
from jax import lax
from jax.experimental import pallas as pl
from jax.experimental.pallas import tpu as pltpu
from jax.experimental.pallas import tpu_sc as plsc

NUM_EXPERTS = 64
HIDDEN = 1024
INTER = 512
TOKENS = 4096
TOP_K = 2

BM = 256
NPAIRS = TOKENS * TOP_K
PADDED = NPAIRS + NUM_EXPERTS * BM
GRID = PADDED // BM

NUM_WORKERS = 32
GATHER_CHUNK = 64


def _routing_metadata(top_k_index):
    e_flat = top_k_index.astype(jnp.int32).reshape(-1)
    iota = jnp.arange(NPAIRS, dtype=jnp.int32)
    e_ids = jnp.arange(NUM_EXPERTS, dtype=jnp.int32)
    e_sorted, order = lax.sort((e_flat, iota), num_keys=1, is_stable=True)
    t_sorted = order // TOP_K

    counts = jnp.sum((e_flat[:, None] == e_ids[None, :]).astype(jnp.int32), axis=0)
    tiles_per_e = (counts + BM - 1) // BM
    padded_sz = tiles_per_e * BM
    pad_start = jnp.cumsum(padded_sz) - padded_sz
    grp_start = jnp.cumsum(counts) - counts
    delta = (pad_start - grp_start).astype(jnp.int32)

    eqs = e_sorted[:, None] == e_ids[None, :]
    pos = iota + jnp.sum(jnp.where(eqs, delta[None, :], 0), axis=1).astype(jnp.int32)

    q = jnp.arange(PADDED, dtype=jnp.int32)
    cum_padded = jnp.cumsum(padded_sz).astype(jnp.int32)
    e_q = jnp.sum((cum_padded[None, :] <= q[:, None]).astype(jnp.int32), axis=1)
    eq_q = e_q[:, None] == e_ids[None, :]
    ps_q = jnp.sum(jnp.where(eq_q, pad_start[None, :], 0), axis=1).astype(jnp.int32)
    gs_q = jnp.sum(jnp.where(eq_q, grp_start[None, :], 0), axis=1).astype(jnp.int32)
    cnt_q = jnp.sum(jnp.where(eq_q, counts[None, :], 0), axis=1).astype(jnp.int32)
    rank_q = q - ps_q
    j = jnp.clip(rank_q + gs_q, 0, NPAIRS - 1)
    t_pad = jnp.where(rank_q < cnt_q, t_sorted[j], q % TOKENS)

    _, pos_by_pair = lax.sort((order, pos), num_keys=1)
    idx_cat = jnp.concatenate([pos_by_pair[0::2], pos_by_pair[1::2]])

    cum_tiles = jnp.cumsum(tiles_per_e).astype(jnp.int32)
    used = jnp.sum(tiles_per_e).astype(jnp.int32)
    g = jnp.arange(GRID, dtype=jnp.int32)
    tile_e = jnp.sum((cum_tiles[None, :] <= g[:, None]).astype(jnp.int32), axis=1)
    last_e = jnp.max(jnp.where(counts > 0, e_ids, 0)).astype(jnp.int32)
    tile_e = jnp.where(g < used, tile_e, last_e)
    tile_b = jnp.where(g < used, g, used - 1)
    return t_pad, idx_cat, tile_e, tile_b


def _gather_body(table_hbm, idx_hbm, out_hbm, idx_v, rows_v, sem, *, per_w):
    wid = lax.axis_index("s") * 2 + lax.axis_index("c")
    base = wid * per_w

    def chunk_body(c, carry):
        b = base + c * GATHER_CHUNK
        pltpu.sync_copy(idx_hbm.at[pl.ds(b, GATHER_CHUNK)], idx_v)
        pltpu.async_copy(table_hbm.at[idx_v], rows_v, sem).wait()
        pltpu.sync_copy(rows_v, out_hbm.at[pl.ds(b, GATHER_CHUNK)])
        return carry

    lax.fori_loop(0, per_w // GATHER_CHUNK, chunk_body, 0)


def _gather_rows(table, idx, nrows):
    width = table.shape[1]
    per_w = nrows // NUM_WORKERS
    mesh = plsc.VectorSubcoreMesh(core_axis_name="c", subcore_axis_name="s")
    k = pl.kernel(
        functools.partial(_gather_body, per_w=per_w),
        out_type=jax.ShapeDtypeStruct((nrows, width), table.dtype),
        mesh=mesh,
        scratch_types=[
            pltpu.VMEM((GATHER_CHUNK,), jnp.int32),
            pltpu.VMEM((GATHER_CHUNK, width), table.dtype),
            pltpu.SemaphoreType.DMA,
        ],
    )
    return k(table, idx)


def _mlp_body(te_ref, tb_ref, x_ref, gu_ref, dn_ref, y_ref):
    i = pl.program_id(0)

    @pl.when(tb_ref[i] == i)
    def _():
        x = x_ref[...]
        gu_w = gu_ref[0]
        gu = lax.dot_general(x, gu_w, (((1,), (1,)), ((), ())),
                             preferred_element_type=jnp.float32)
        gate = gu[:, :INTER]
        up = gu[:, INTER:]
        h = gate * jax.nn.sigmoid(gate) * up
        dn_w = dn_ref[0]
        y_ref[...] = lax.dot_general(h, dn_w, (((1,), (1,)), ((), ())),
                                     preferred_element_type=jnp.float32)


def _grouped_mlp(tile_e, tile_b, x, gate_up_proj, down_proj):
    grid_spec = pltpu.PrefetchScalarGridSpec(
        num_scalar_prefetch=2,
        grid=(GRID,),
        in_specs=[
            pl.BlockSpec((BM, HIDDEN), lambda i, te, tb: (tb[i], 0)),
            pl.BlockSpec((1, 2 * INTER, HIDDEN), lambda i, te, tb: (te[i], 0, 0)),
            pl.BlockSpec((1, HIDDEN, INTER), lambda i, te, tb: (te[i], 0, 0)),
        ],
        out_specs=pl.BlockSpec((BM, HIDDEN), lambda i, te, tb: (tb[i], 0)),
    )
    return pl.pallas_call(
        _mlp_body,
        grid_spec=grid_spec,
        out_shape=jax.ShapeDtypeStruct((PADDED, HIDDEN), jnp.float32),
    )(tile_e, tile_b, x, gate_up_proj, down_proj)


def _combine_body(y0_ref, y1_ref, w0_ref, w1_ref, o_ref):
    o_ref[...] = y0_ref[...] * w0_ref[...] + y1_ref[...] * w1_ref[...]


_COMBINE_RB = 256


def _combine(yp, w0c, w1c):
    nb = TOKENS // _COMBINE_RB
    return pl.pallas_call(
        _combine_body,
        grid=(nb,),
        in_specs=[
            pl.BlockSpec((_COMBINE_RB, HIDDEN), lambda i: (i, 0)),
            pl.BlockSpec((_COMBINE_RB, HIDDEN), lambda i: (i + nb, 0)),
            pl.BlockSpec((_COMBINE_RB, 1), lambda i: (i, 0)),
            pl.BlockSpec((_COMBINE_RB, 1), lambda i: (i, 0)),
        ],
        out_specs=pl.BlockSpec((_COMBINE_RB, HIDDEN), lambda i: (i, 0)),
        out_shape=jax.ShapeDtypeStruct((TOKENS, HIDDEN), jnp.float32),
    )(yp, yp, w0c, w1c)


def kernel(hidden_states, top_k_index, top_k_weights, gate_up_proj, down_proj):
    t_pad, idx_cat, tile_e, tile_b = _routing_metadata(top_k_index)
    x = _gather_rows(hidden_states, t_pad, PADDED)
    y = _grouped_mlp(tile_e, tile_b, x, gate_up_proj, down_proj)
    yp = _gather_rows(y, idx_cat, 2 * TOKENS)
    w0c = top_k_weights[:, 0:1]
    w1c = top_k_weights[:, 1:2]
    return _combine(yp, w0c, w1c)


# speedup vs baseline: 1.4520x; 1.4520x over previous
import functools

import jax
import jax.numpy as jnp
from jax import lax
from jax.experimental import pallas as pl
from jax.experimental.pallas import tpu as pltpu
from jax.experimental.pallas import tpu_sc as plsc

NUM_EXPERTS = 64
HIDDEN = 1024
INTER = 512
TOKENS = 4096
TOP_K = 2

BM = 256
NPAIRS = TOKENS * TOP_K
PADDED = NPAIRS + NUM_EXPERTS * BM
GRID = PADDED // BM

NUM_WORKERS = 32
GATHER_CHUNK = 64


def _routing_metadata(top_k_index):
    e_flat = top_k_index.astype(jnp.int32).reshape(-1)
    iota = jnp.arange(NPAIRS, dtype=jnp.int32)
    e_ids = jnp.arange(NUM_EXPERTS, dtype=jnp.int32)
    e_sorted, order = lax.sort((e_flat, iota), num_keys=1, is_stable=True)
    t_sorted = order // TOP_K

    counts = jnp.sum((e_flat[:, None] == e_ids[None, :]).astype(jnp.int32), axis=0)
    tiles_per_e = (counts + BM - 1) // BM
    padded_sz = tiles_per_e * BM
    pad_start = jnp.cumsum(padded_sz) - padded_sz
    grp_start = jnp.cumsum(counts) - counts
    delta = (pad_start - grp_start).astype(jnp.int32)

    eqs = e_sorted[:, None] == e_ids[None, :]
    pos = iota + jnp.sum(jnp.where(eqs, delta[None, :], 0), axis=1).astype(jnp.int32)

    t_fill = jnp.arange(PADDED, dtype=jnp.int32) % TOKENS
    t_pad = t_fill.at[pos].set(t_sorted)

    _, pos_by_pair = lax.sort((order, pos), num_keys=1)
    idx_cat = jnp.concatenate([pos_by_pair[0::2], pos_by_pair[1::2]])

    cum_tiles = jnp.cumsum(tiles_per_e).astype(jnp.int32)
    used = jnp.sum(tiles_per_e).astype(jnp.int32)
    g = jnp.arange(GRID, dtype=jnp.int32)
    tile_e = jnp.sum((cum_tiles[None, :] <= g[:, None]).astype(jnp.int32), axis=1)
    last_e = jnp.max(jnp.where(counts > 0, e_ids, 0)).astype(jnp.int32)
    tile_e = jnp.where(g < used, tile_e, last_e)
    tile_b = jnp.where(g < used, g, used - 1)
    return t_pad, idx_cat, tile_e, tile_b


def _gather_body(table_hbm, idx_hbm, out_hbm, idx_v, rows_v, sem, *, per_w):
    wid = lax.axis_index("s") * 2 + lax.axis_index("c")
    base = wid * per_w

    def chunk_body(c, carry):
        b = base + c * GATHER_CHUNK
        pltpu.sync_copy(idx_hbm.at[pl.ds(b, GATHER_CHUNK)], idx_v)
        pltpu.async_copy(table_hbm.at[idx_v], rows_v, sem).wait()
        pltpu.sync_copy(rows_v, out_hbm.at[pl.ds(b, GATHER_CHUNK)])
        return carry

    lax.fori_loop(0, per_w // GATHER_CHUNK, chunk_body, 0)


def _gather_rows(table, idx, nrows):
    width = table.shape[1]
    per_w = nrows // NUM_WORKERS
    mesh = plsc.VectorSubcoreMesh(core_axis_name="c", subcore_axis_name="s")
    k = pl.kernel(
        functools.partial(_gather_body, per_w=per_w),
        out_type=jax.ShapeDtypeStruct((nrows, width), table.dtype),
        mesh=mesh,
        scratch_types=[
            pltpu.VMEM((GATHER_CHUNK,), jnp.int32),
            pltpu.VMEM((GATHER_CHUNK, width), table.dtype),
            pltpu.SemaphoreType.DMA,
        ],
    )
    return k(table, idx)


def _mlp_body(te_ref, tb_ref, x_ref, gu_ref, dn_ref, y_ref):
    i = pl.program_id(0)

    @pl.when(tb_ref[i] == i)
    def _():
        x = x_ref[...]
        gu_w = gu_ref[0]
        gu = lax.dot_general(x, gu_w, (((1,), (1,)), ((), ())),
                             preferred_element_type=jnp.float32)
        gate = gu[:, :INTER]
        up = gu[:, INTER:]
        h = gate * jax.nn.sigmoid(gate) * up
        dn_w = dn_ref[0]
        y_ref[...] = lax.dot_general(h, dn_w, (((1,), (1,)), ((), ())),
                                     preferred_element_type=jnp.float32)


def _grouped_mlp(tile_e, tile_b, x, gate_up_proj, down_proj):
    grid_spec = pltpu.PrefetchScalarGridSpec(
        num_scalar_prefetch=2,
        grid=(GRID,),
        in_specs=[
            pl.BlockSpec((BM, HIDDEN), lambda i, te, tb: (tb[i], 0)),
            pl.BlockSpec((1, 2 * INTER, HIDDEN), lambda i, te, tb: (te[i], 0, 0)),
            pl.BlockSpec((1, HIDDEN, INTER), lambda i, te, tb: (te[i], 0, 0)),
        ],
        out_specs=pl.BlockSpec((BM, HIDDEN), lambda i, te, tb: (tb[i], 0)),
    )
    return pl.pallas_call(
        _mlp_body,
        grid_spec=grid_spec,
        out_shape=jax.ShapeDtypeStruct((PADDED, HIDDEN), jnp.float32),
    )(tile_e, tile_b, x, gate_up_proj, down_proj)


def _combine_body(y0_ref, y1_ref, w0_ref, w1_ref, o_ref):
    o_ref[...] = y0_ref[...] * w0_ref[...] + y1_ref[...] * w1_ref[...]


_COMBINE_RB = 256


def _combine(yp, w0c, w1c):
    nb = TOKENS // _COMBINE_RB
    return pl.pallas_call(
        _combine_body,
        grid=(nb,),
        in_specs=[
            pl.BlockSpec((_COMBINE_RB, HIDDEN), lambda i: (i, 0)),
            pl.BlockSpec((_COMBINE_RB, HIDDEN), lambda i: (i + nb, 0)),
            pl.BlockSpec((_COMBINE_RB, 1), lambda i: (i, 0)),
            pl.BlockSpec((_COMBINE_RB, 1), lambda i: (i, 0)),
        ],
        out_specs=pl.BlockSpec((_COMBINE_RB, HIDDEN), lambda i: (i, 0)),
        out_shape=jax.ShapeDtypeStruct((TOKENS, HIDDEN), jnp.float32),
    )(yp, yp, w0c, w1c)


def kernel(hidden_states, top_k_index, top_k_weights, gate_up_proj, down_proj):
    t_pad, idx_cat, tile_e, tile_b = _routing_metadata(top_k_index)
    x = _gather_rows(hidden_states, t_pad, PADDED)
    y = _grouped_mlp(tile_e, tile_b, x, gate_up_proj, down_proj)
    yp = _gather_rows(y, idx_cat, 2 * TOKENS)
    w0c = top_k_weights[:, 0:1]
    w1c = top_k_weights[:, 1:2]
    return _combine(yp, w0c, w1c)


# speedup vs baseline: 1.8883x; 1.3005x over previous
import functools

import jax
import jax.numpy as jnp
from jax import lax
from jax.experimental import pallas as pl
from jax.experimental.pallas import tpu as pltpu
from jax.experimental.pallas import tpu_sc as plsc

NUM_EXPERTS = 64
HIDDEN = 1024
INTER = 512
TOKENS = 4096
TOP_K = 2

BM = 256
NPAIRS = TOKENS * TOP_K
PADDED = NPAIRS + NUM_EXPERTS * BM
GRID = PADDED // BM

NUM_WORKERS = 32
GATHER_CHUNK = 64


def _routing_metadata(top_k_index):
    e_flat = top_k_index.astype(jnp.int32).reshape(-1)
    iota = jnp.arange(NPAIRS, dtype=jnp.int32)
    e_ids = jnp.arange(NUM_EXPERTS, dtype=jnp.int32)
    e_sorted, order = lax.sort((e_flat, iota), num_keys=1, is_stable=True)
    t_sorted = order // TOP_K

    counts = jnp.sum((e_flat[:, None] == e_ids[None, :]).astype(jnp.int32), axis=0)
    tiles_per_e = (counts + BM - 1) // BM
    padded_sz = tiles_per_e * BM
    pad_start = jnp.cumsum(padded_sz) - padded_sz
    grp_start = jnp.cumsum(counts) - counts
    delta = (pad_start - grp_start).astype(jnp.int32)

    eqs = e_sorted[:, None] == e_ids[None, :]
    pos = iota + jnp.sum(jnp.where(eqs, delta[None, :], 0), axis=1).astype(jnp.int32)

    _, pos_by_pair = lax.sort((order, pos), num_keys=1)
    idx_cat = jnp.concatenate([pos_by_pair[0::2], pos_by_pair[1::2]])

    cum_tiles = jnp.cumsum(tiles_per_e).astype(jnp.int32)
    used = jnp.sum(tiles_per_e).astype(jnp.int32)
    g = jnp.arange(GRID, dtype=jnp.int32)
    tile_e = jnp.sum((cum_tiles[None, :] <= g[:, None]).astype(jnp.int32), axis=1)
    last_e = jnp.max(jnp.where(counts > 0, e_ids, 0)).astype(jnp.int32)
    tile_e = jnp.where(g < used, tile_e, last_e)
    tile_b = jnp.where(g < used, g, used - 1)
    return t_sorted, pos, idx_cat, tile_e, tile_b


def _dispatch_body(hidden_hbm, tsrt_hbm, pos_hbm, x_hbm, tidx_v, pidx_v, rows_v, sem):
    wid = lax.axis_index("s") * 2 + lax.axis_index("c")
    base = wid * (NPAIRS // NUM_WORKERS)

    def chunk_body(c, carry):
        b = base + c * GATHER_CHUNK
        pltpu.sync_copy(tsrt_hbm.at[pl.ds(b, GATHER_CHUNK)], tidx_v)
        pltpu.sync_copy(pos_hbm.at[pl.ds(b, GATHER_CHUNK)], pidx_v)
        pltpu.async_copy(hidden_hbm.at[tidx_v], rows_v, sem).wait()
        pltpu.async_copy(rows_v, x_hbm.at[pidx_v], sem).wait()
        return carry

    lax.fori_loop(0, NPAIRS // NUM_WORKERS // GATHER_CHUNK, chunk_body, 0)


def _dispatch_x(hidden_states, t_sorted, pos):
    mesh = plsc.VectorSubcoreMesh(core_axis_name="c", subcore_axis_name="s")
    k = pl.kernel(
        _dispatch_body,
        out_type=jax.ShapeDtypeStruct((PADDED, HIDDEN), jnp.float32),
        mesh=mesh,
        scratch_types=[
            pltpu.VMEM((GATHER_CHUNK,), jnp.int32),
            pltpu.VMEM((GATHER_CHUNK,), jnp.int32),
            pltpu.VMEM((GATHER_CHUNK, HIDDEN), jnp.float32),
            pltpu.SemaphoreType.DMA,
        ],
    )
    return k(hidden_states, t_sorted, pos)


def _gather_body(table_hbm, idx_hbm, out_hbm, idx_v, rows_v, sem, *, per_w):
    wid = lax.axis_index("s") * 2 + lax.axis_index("c")
    base = wid * per_w

    def chunk_body(c, carry):
        b = base + c * GATHER_CHUNK
        pltpu.sync_copy(idx_hbm.at[pl.ds(b, GATHER_CHUNK)], idx_v)
        pltpu.async_copy(table_hbm.at[idx_v], rows_v, sem).wait()
        pltpu.sync_copy(rows_v, out_hbm.at[pl.ds(b, GATHER_CHUNK)])
        return carry

    lax.fori_loop(0, per_w // GATHER_CHUNK, chunk_body, 0)


def _gather_rows(table, idx, nrows):
    width = table.shape[1]
    per_w = nrows // NUM_WORKERS
    mesh = plsc.VectorSubcoreMesh(core_axis_name="c", subcore_axis_name="s")
    k = pl.kernel(
        functools.partial(_gather_body, per_w=per_w),
        out_type=jax.ShapeDtypeStruct((nrows, width), table.dtype),
        mesh=mesh,
        scratch_types=[
            pltpu.VMEM((GATHER_CHUNK,), jnp.int32),
            pltpu.VMEM((GATHER_CHUNK, width), table.dtype),
            pltpu.SemaphoreType.DMA,
        ],
    )
    return k(table, idx)


def _mlp_body(te_ref, tb_ref, x_ref, gu_ref, dn_ref, y_ref):
    i = pl.program_id(0)

    @pl.when(tb_ref[i] == i)
    def _():
        x = x_ref[...]
        gu_w = gu_ref[0]
        gu = lax.dot_general(x, gu_w, (((1,), (1,)), ((), ())),
                             preferred_element_type=jnp.float32)
        gate = gu[:, :INTER]
        up = gu[:, INTER:]
        h = gate * jax.nn.sigmoid(gate) * up
        dn_w = dn_ref[0]
        y_ref[...] = lax.dot_general(h, dn_w, (((1,), (1,)), ((), ())),
                                     preferred_element_type=jnp.float32)


def _grouped_mlp(tile_e, tile_b, x, gate_up_proj, down_proj):
    grid_spec = pltpu.PrefetchScalarGridSpec(
        num_scalar_prefetch=2,
        grid=(GRID,),
        in_specs=[
            pl.BlockSpec((BM, HIDDEN), lambda i, te, tb: (tb[i], 0)),
            pl.BlockSpec((1, 2 * INTER, HIDDEN), lambda i, te, tb: (te[i], 0, 0)),
            pl.BlockSpec((1, HIDDEN, INTER), lambda i, te, tb: (te[i], 0, 0)),
        ],
        out_specs=pl.BlockSpec((BM, HIDDEN), lambda i, te, tb: (tb[i], 0)),
    )
    return pl.pallas_call(
        _mlp_body,
        grid_spec=grid_spec,
        out_shape=jax.ShapeDtypeStruct((PADDED, HIDDEN), jnp.float32),
    )(tile_e, tile_b, x, gate_up_proj, down_proj)


def _combine_body(y0_ref, y1_ref, w0_ref, w1_ref, o_ref):
    o_ref[...] = y0_ref[...] * w0_ref[...] + y1_ref[...] * w1_ref[...]


_COMBINE_RB = 256


def _combine(yp, w0c, w1c):
    nb = TOKENS // _COMBINE_RB
    return pl.pallas_call(
        _combine_body,
        grid=(nb,),
        in_specs=[
            pl.BlockSpec((_COMBINE_RB, HIDDEN), lambda i: (i, 0)),
            pl.BlockSpec((_COMBINE_RB, HIDDEN), lambda i: (i + nb, 0)),
            pl.BlockSpec((_COMBINE_RB, 1), lambda i: (i, 0)),
            pl.BlockSpec((_COMBINE_RB, 1), lambda i: (i, 0)),
        ],
        out_specs=pl.BlockSpec((_COMBINE_RB, HIDDEN), lambda i: (i, 0)),
        out_shape=jax.ShapeDtypeStruct((TOKENS, HIDDEN), jnp.float32),
    )(yp, yp, w0c, w1c)


def kernel(hidden_states, top_k_index, top_k_weights, gate_up_proj, down_proj):
    t_sorted, pos, idx_cat, tile_e, tile_b = _routing_metadata(top_k_index)
    x = _dispatch_x(hidden_states, t_sorted, pos)
    y = _grouped_mlp(tile_e, tile_b, x, gate_up_proj, down_proj)
    yp = _gather_rows(y, idx_cat, 2 * TOKENS)
    w0c = top_k_weights[:, 0:1]
    w1c = top_k_weights[:, 1:2]
    return _combine(yp, w0c, w1c)
